# Initial kernel scaffold; baseline (speedup 1.0000x reference)
import jax
import jax.numpy as jnp
from jax.experimental import pallas as pl


def kernel(x, edge_index_a, edge_index_b, W0a, b0a, W0b, b0b, W1a, b1a, W1b, b1b, Wh1, bh1, Wh2, bh2):
    raise NotImplementedError("write your pallas kernel here")


# speedup vs baseline: 5.6721x; 5.6721x over previous
import functools

import jax
import jax.numpy as jnp
from jax import lax
from jax.experimental import pallas as pl
from jax.experimental.pallas import tpu as pltpu
from jax.experimental.pallas import tpu_sc as plsc

N = 10000
D = 128
E = 320000


NSUB = 16
CH = 128
NCHT = E // CH
WB = 640
ZR = 160

@functools.cache
def _seg2_built():
    mesh = plsc.VectorSubcoreMesh(core_axis_name="c", subcore_axis_name="s")
    return functools.partial(
        pl.kernel,
        mesh=mesh,
        out_type=(
            jax.ShapeDtypeStruct((N, D), jnp.float32),
            jax.ShapeDtypeStruct((N, D), jnp.float32),
        ),
        scratch_types=[
            pltpu.VMEM((2, CH), jnp.int32),
            pltpu.VMEM((CH, D), jnp.float32),
            pltpu.VMEM((ZR, D), jnp.float32),
            pltpu.VMEM_SHARED((N, D), jnp.float32),
            pltpu.SemaphoreType.DMA,
        ],
    )(_seg2_body)


def _seg2(ha, hb, ea, eb):
    return _seg2_built()(ha, hb, ea, eb)


def _seg2_body(ha, hb, ea, eb, oa, ob, idx2, rows, zbuf, accum, sem):
    c = lax.axis_index("c")
    s = lax.axis_index("s")

    zv = jnp.zeros((16,), jnp.float32)

    def zrow(i, carry):
        for j in range(D // 16):
            zbuf[i, pl.ds(j * 16, 16)] = zv
        return carry

    lax.fori_loop(0, ZR, zrow, 0)
    base_r = jnp.minimum(s * WB, N - WB)
    for k in range(WB // ZR):
        pltpu.sync_copy(zbuf, accum.at[pl.ds(base_r + k * ZR, ZR)])
    plsc.subcore_barrier()

    nch = NCHT // NSUB + jnp.where(s < NCHT % NSUB, 1, 0)

    def run(h_ref, e_ref):
        def chunk(i, carry):
            off = (s + NSUB * i) * CH
            pltpu.sync_copy(e_ref.at[:, pl.ds(off, CH)], idx2)
            pltpu.async_copy(h_ref.at[idx2.at[0]], rows, sem).wait()
            pltpu.sync_copy(rows, accum.at[idx2.at[1]], add=True)
            return carry

        lax.fori_loop(0, nch, chunk, 0)

    @pl.when(c == 0)
    def _():
        run(ha, ea)

    @pl.when(c == 1)
    def _():
        run(hb, eb)

    plsc.subcore_barrier()

    @pl.when(c == 0)
    def _():
        pltpu.sync_copy(accum.at[pl.ds(base_r, WB)], oa.at[pl.ds(base_r, WB)])

    @pl.when(c == 1)
    def _():
        pltpu.sync_copy(accum.at[pl.ds(base_r, WB)], ob.at[pl.ds(base_r, WB)])


RB = 1000
GRID = N // RB

_row_spec = pl.BlockSpec((RB, D), lambda r: (r, 0))
_w_spec = pl.BlockSpec((D, D), lambda r: (0, 0))
_b_spec = pl.BlockSpec((1, D), lambda r: (0, 0))
_row_shape = jax.ShapeDtypeStruct((N, D), jnp.float32)

_INV_SQRT2 = 0.7071067811865476


def _gelu(t):
    return 0.5 * t * (1.0 + lax.erf(t * _INV_SQRT2))


def _mm2_body(x_ref, wa_ref, wb_ref, oa_ref, ob_ref):
    xb = x_ref[...]
    oa_ref[...] = jnp.dot(xb, wa_ref[...], preferred_element_type=jnp.float32)
    ob_ref[...] = jnp.dot(xb, wb_ref[...], preferred_element_type=jnp.float32)


def _mm2(x, wa, wb):
    return pl.pallas_call(
        _mm2_body,
        grid=(GRID,),
        in_specs=[_row_spec, _w_spec, _w_spec],
        out_specs=[_row_spec, _row_spec],
        out_shape=[_row_shape, _row_shape],
    )(x, wa, wb)


def _comb_body(aa_ref, ab_ref, ha_ref, hb_ref, ba_ref, bb_ref,
               wa_ref, wb_ref, oa_ref, ob_ref):
    t = (aa_ref[...] + ab_ref[...] + ha_ref[...] + hb_ref[...]
         + ba_ref[...] + bb_ref[...])
    h = _gelu(t)
    oa_ref[...] = jnp.dot(h, wa_ref[...], preferred_element_type=jnp.float32)
    ob_ref[...] = jnp.dot(h, wb_ref[...], preferred_element_type=jnp.float32)


def _comb_mm2(aa, ab, ha, hb, ba, bb, wa, wb):
    return pl.pallas_call(
        _comb_body,
        grid=(GRID,),
        in_specs=[_row_spec, _row_spec, _row_spec, _row_spec,
                  _b_spec, _b_spec, _w_spec, _w_spec],
        out_specs=[_row_spec, _row_spec],
        out_shape=[_row_shape, _row_shape],
    )(aa, ab, ha, hb, ba, bb, wa, wb)


def _head_body(aa_ref, ab_ref, ha_ref, hb_ref, ba_ref, bb_ref,
               w1_ref, b1_ref, w2_ref, b2_ref, o_ref):
    t = (aa_ref[...] + ab_ref[...] + ha_ref[...] + hb_ref[...]
         + ba_ref[...] + bb_ref[...])
    h = _gelu(t)
    h = _gelu(jnp.dot(h, w1_ref[...], preferred_element_type=jnp.float32)
              + b1_ref[...])
    o_ref[...] = (jnp.dot(h, w2_ref[...], preferred_element_type=jnp.float32)
                  + b2_ref[...])


def _head(aa, ab, ha, hb, ba, bb, w1, b1, w2, b2):
    return pl.pallas_call(
        _head_body,
        grid=(GRID,),
        in_specs=[_row_spec, _row_spec, _row_spec, _row_spec,
                  _b_spec, _b_spec, _w_spec, _b_spec, _w_spec, _b_spec],
        out_specs=_row_spec,
        out_shape=_row_shape,
    )(aa, ab, ha, hb, ba, bb, w1, b1, w2, b2)


def kernel(x, edge_index_a, edge_index_b,
           W0a, b0a, W0b, b0b, W1a, b1a, W1b, b1b,
           Wh1, bh1, Wh2, bh2):
    ha, hb = _mm2(x, W0a, W0b)
    aa, ab = _seg2(ha, hb, edge_index_a, edge_index_b)
    h1a, h1b = _comb_mm2(aa, ab, ha, hb,
                         b0a.reshape(1, D), b0b.reshape(1, D), W1a, W1b)
    a1a, a1b = _seg2(h1a, h1b, edge_index_a, edge_index_b)
    out = _head(a1a, a1b, h1a, h1b,
                b1a.reshape(1, D), b1b.reshape(1, D),
                Wh1, bh1.reshape(1, D), Wh2, bh2.reshape(1, D))
    return out
